# baseline (speedup 1.0000x reference)
import functools

import jax
import jax.numpy as jnp
from jax import lax
from jax.experimental import pallas as pl
from jax.experimental.pallas import tpu as pltpu
from jax.experimental.pallas import tpu_sc as plsc

N = 256
M = 100000
B = 16384


_CHUNK = 1024
_KSTEPS = (M + _CHUNK - 1) // _CHUNK
_NEG = -1e30


def _lse_transpose_body(w_ref, wt_ref, lse_ref, m_ref, s_ref):
    i = pl.program_id(0)

    @pl.when(i == 0)
    def _init():
        m_ref[...] = jnp.full((N, 1), _NEG, jnp.float32)
        s_ref[...] = jnp.zeros((N, 1), jnp.float32)

    w = w_ref[...]
    wt_ref[...] = w.T

    col = i * _CHUNK + lax.broadcasted_iota(jnp.int32, (N, _CHUNK), 1)
    x = jnp.where(col < M, w, _NEG)

    m_prev = m_ref[...]
    m_new = jnp.maximum(m_prev, jnp.max(x, axis=1, keepdims=True))
    s_ref[...] = (s_ref[...] * jnp.exp(m_prev - m_new)
                  + jnp.sum(jnp.exp(x - m_new), axis=1, keepdims=True))
    m_ref[...] = m_new

    @pl.when(i == _KSTEPS - 1)
    def _fin():
        lse_ref[...] = m_ref[...] + jnp.log(s_ref[...])


def _lse_transpose(w):
    return pl.pallas_call(
        _lse_transpose_body,
        grid=(_KSTEPS,),
        in_specs=[pl.BlockSpec((N, _CHUNK), lambda i: (0, i))],
        out_specs=[
            pl.BlockSpec((_CHUNK, N), lambda i: (i, 0)),
            pl.BlockSpec((N, 1), lambda i: (0, 0)),
        ],
        out_shape=[
            jax.ShapeDtypeStruct((M, N), jnp.float32),
            jax.ShapeDtypeStruct((N, 1), jnp.float32),
        ],
        scratch_shapes=[
            pltpu.VMEM((N, 1), jnp.float32),
            pltpu.VMEM((N, 1), jnp.float32),
        ],
    )(w)


_NC, _NS = 2, 16
_NW = _NC * _NS
_BPW = B // _NW
_GCHUNK = 128


def _gather_body(wt_hbm, idx_hbm, out_hbm, idx_v, rows_v, sem):
    wid = lax.axis_index("s") * _NC + lax.axis_index("c")
    base = wid * _BPW
    pltpu.sync_copy(idx_hbm.at[pl.ds(base, _BPW)], idx_v)
    for c in range(_BPW // _GCHUNK):
        pltpu.async_copy(
            wt_hbm.at[idx_v.at[pl.ds(c * _GCHUNK, _GCHUNK)]], rows_v, sem
        ).wait()
        pltpu.sync_copy(rows_v, out_hbm.at[pl.ds(base + c * _GCHUNK, _GCHUNK)])


def _sc_gather(wt, idx):
    mesh = plsc.VectorSubcoreMesh(core_axis_name="c", subcore_axis_name="s")
    k = functools.partial(
        pl.kernel,
        mesh=mesh,
        out_type=jax.ShapeDtypeStruct((B, N), jnp.float32),
        scratch_types=[
            pltpu.VMEM((_BPW,), jnp.int32),
            pltpu.VMEM((_GCHUNK, N), jnp.float32),
            pltpu.SemaphoreType.DMA,
        ],
    )(_gather_body)
    return k(wt, idx)


_CBLK = 2048


def _sub_body(g_ref, lse_ref, o_ref):
    o_ref[...] = g_ref[...] - lse_ref[...]


def _sub_lse(g, lse_row):
    return pl.pallas_call(
        _sub_body,
        grid=(B // _CBLK,),
        in_specs=[
            pl.BlockSpec((_CBLK, N), lambda i: (i, 0)),
            pl.BlockSpec((1, N), lambda i: (0, 0)),
        ],
        out_specs=pl.BlockSpec((_CBLK, N), lambda i: (i, 0)),
        out_shape=jax.ShapeDtypeStruct((B, N), jnp.float32),
    )(g, lse_row)


def kernel(x_t, W):
    idx = x_t.astype(jnp.int32)
    wt, lse = _lse_transpose(W)
    g = _sc_gather(wt, idx)
    lse_row = lse.reshape(1, N)
    return _sub_lse(g, lse_row)


# speedup vs baseline: 1.0929x; 1.0929x over previous
import functools

import jax
import jax.numpy as jnp
from jax import lax
from jax.experimental import pallas as pl
from jax.experimental.pallas import tpu as pltpu
from jax.experimental.pallas import tpu_sc as plsc

N = 256
M = 100000
B = 16384


_CHUNK = 2048
_KSTEPS = (M + _CHUNK - 1) // _CHUNK
_NEG = -1e30


def _lse_transpose_body(w_ref, eye_ref, wt_ref, lse_ref, m_ref, s_ref):
    i = pl.program_id(0)

    @pl.when(i == 0)
    def _init():
        m_ref[...] = jnp.full((N, 1), _NEG, jnp.float32)
        s_ref[...] = jnp.zeros((N, 1), jnp.float32)

    w = w_ref[...]
    wb = w.astype(jnp.bfloat16)
    wt_ref[...] = lax.dot_general(
        wb, eye_ref[...],
        dimension_numbers=(((0,), (0,)), ((), ())),
        preferred_element_type=jnp.float32,
    )

    def _update(x):
        m_prev = m_ref[...]
        m_new = jnp.maximum(m_prev, jnp.max(x, axis=1, keepdims=True))
        s_ref[...] = (s_ref[...] * jnp.exp(m_prev - m_new)
                      + jnp.sum(jnp.exp(x - m_new), axis=1, keepdims=True))
        m_ref[...] = m_new

    @pl.when(i < _KSTEPS - 1)
    def _full():
        _update(w)

    @pl.when(i == _KSTEPS - 1)
    def _last():
        col = i * _CHUNK + lax.broadcasted_iota(jnp.int32, (N, _CHUNK), 1)
        _update(jnp.where(col < M, w, _NEG))
        lse_ref[...] = m_ref[...] + jnp.log(s_ref[...])


def _lse_transpose(w, eye):
    return pl.pallas_call(
        _lse_transpose_body,
        grid=(_KSTEPS,),
        in_specs=[
            pl.BlockSpec((N, _CHUNK), lambda i: (0, i)),
            pl.BlockSpec((N, N), lambda i: (0, 0)),
        ],
        out_specs=[
            pl.BlockSpec((_CHUNK, N), lambda i: (i, 0)),
            pl.BlockSpec((N, 1), lambda i: (0, 0)),
        ],
        out_shape=[
            jax.ShapeDtypeStruct((M, N), jnp.float32),
            jax.ShapeDtypeStruct((N, 1), jnp.float32),
        ],
        scratch_shapes=[
            pltpu.VMEM((N, 1), jnp.float32),
            pltpu.VMEM((N, 1), jnp.float32),
        ],
    )(w, eye)


_NC, _NS = 2, 16
_NW = _NC * _NS
_BPW = B // _NW
_GCHUNK = 128


def _gather_body(wt_hbm, idx_hbm, out_hbm, idx_v, rows_v, sem):
    wid = lax.axis_index("s") * _NC + lax.axis_index("c")
    base = wid * _BPW
    pltpu.sync_copy(idx_hbm.at[pl.ds(base, _BPW)], idx_v)
    for c in range(_BPW // _GCHUNK):
        pltpu.async_copy(
            wt_hbm.at[idx_v.at[pl.ds(c * _GCHUNK, _GCHUNK)]], rows_v, sem
        ).wait()
        pltpu.sync_copy(rows_v, out_hbm.at[pl.ds(base + c * _GCHUNK, _GCHUNK)])


def _sc_gather(wt, idx):
    mesh = plsc.VectorSubcoreMesh(core_axis_name="c", subcore_axis_name="s")
    k = functools.partial(
        pl.kernel,
        mesh=mesh,
        out_type=jax.ShapeDtypeStruct((B, N), jnp.float32),
        scratch_types=[
            pltpu.VMEM((_BPW,), jnp.int32),
            pltpu.VMEM((_GCHUNK, N), jnp.float32),
            pltpu.SemaphoreType.DMA,
        ],
    )(_gather_body)
    return k(wt, idx)


_CBLK = 2048


def _sub_body(g_ref, lse_ref, o_ref):
    o_ref[...] = g_ref[...] - lse_ref[...]


def _sub_lse(g, lse_row):
    return pl.pallas_call(
        _sub_body,
        grid=(B // _CBLK,),
        in_specs=[
            pl.BlockSpec((_CBLK, N), lambda i: (i, 0)),
            pl.BlockSpec((1, N), lambda i: (0, 0)),
        ],
        out_specs=pl.BlockSpec((_CBLK, N), lambda i: (i, 0)),
        out_shape=jax.ShapeDtypeStruct((B, N), jnp.float32),
    )(g, lse_row)


def kernel(x_t, W):
    idx = x_t.astype(jnp.int32)
    eye = jnp.eye(N, dtype=jnp.bfloat16)
    wt, lse = _lse_transpose(W, eye)
    g = _sc_gather(wt, idx)
    lse_row = lse.reshape(1, N)
    return _sub_lse(g, lse_row)


# speedup vs baseline: 1.2136x; 1.1105x over previous
import functools

import jax
import jax.numpy as jnp
from jax import lax
from jax.experimental import pallas as pl
from jax.experimental.pallas import tpu as pltpu
from jax.experimental.pallas import tpu_sc as plsc

N = 256
M = 100000
B = 16384


_CHUNK = 2048
_NCH = 48
_TAIL0 = _NCH * _CHUNK
_NBUF = 5
_NEG = -1e30


def _in_copy(w_hbm, inbuf, insem, g, slot):
    return pltpu.make_async_copy(
        w_hbm.at[:, pl.ds(g * _CHUNK, _CHUNK)], inbuf.at[slot], insem.at[slot]
    )


def _out_copy(wt_hbm, outbuf, outsem, g, slot):
    return pltpu.make_async_copy(
        outbuf.at[slot], wt_hbm.at[pl.ds(g * _CHUNK, _CHUNK), :], outsem.at[slot]
    )


def _main_body(w_hbm, eye_ref, wt_hbm, m_out, s_out,
               inbuf, outbuf, insem, outsem, m_ref, s_ref):
    m_ref[...] = jnp.full((N, 1), _NEG, jnp.float32)
    s_ref[...] = jnp.zeros((N, 1), jnp.float32)

    for g in range(_NBUF):
        _in_copy(w_hbm, inbuf, insem, g, g).start()

    def body(g, carry):
        slot = lax.rem(g, _NBUF)
        _in_copy(w_hbm, inbuf, insem, g, slot).wait()
        w = inbuf[slot]

        @pl.when(g >= _NBUF)
        def _drain_out():
            _out_copy(wt_hbm, outbuf, outsem, g - _NBUF, slot).wait()

        wb = w.astype(jnp.bfloat16)
        outbuf[slot] = lax.dot_general(
            wb, eye_ref[...],
            dimension_numbers=(((0,), (0,)), ((), ())),
            preferred_element_type=jnp.float32,
        )
        _out_copy(wt_hbm, outbuf, outsem, g, slot).start()

        @pl.when(g + _NBUF < _NCH)
        def _refill():
            _in_copy(w_hbm, inbuf, insem, g + _NBUF, slot).start()

        m_prev = m_ref[...]
        m_new = jnp.maximum(m_prev, jnp.max(w, axis=1, keepdims=True))
        s_ref[...] = (s_ref[...] * jnp.exp(m_prev - m_new)
                      + jnp.sum(jnp.exp(w - m_new), axis=1, keepdims=True))
        m_ref[...] = m_new
        return carry

    lax.fori_loop(0, _NCH, body, 0)

    for k in range(_NBUF):
        g = _NCH - _NBUF + k
        _out_copy(wt_hbm, outbuf, outsem, g, g % _NBUF).wait()

    m_out[...] = m_ref[...]
    s_out[...] = s_ref[...]


def _main_pass(w, eye):
    return pl.pallas_call(
        _main_body,
        in_specs=[
            pl.BlockSpec(memory_space=pl.ANY),
            pl.BlockSpec(memory_space=pltpu.VMEM),
        ],
        out_specs=[
            pl.BlockSpec(memory_space=pl.ANY),
            pl.BlockSpec(memory_space=pltpu.VMEM),
            pl.BlockSpec(memory_space=pltpu.VMEM),
        ],
        out_shape=[
            jax.ShapeDtypeStruct((M, N), jnp.float32),
            jax.ShapeDtypeStruct((N, 1), jnp.float32),
            jax.ShapeDtypeStruct((N, 1), jnp.float32),
        ],
        scratch_shapes=[
            pltpu.VMEM((_NBUF, N, _CHUNK), jnp.float32),
            pltpu.VMEM((_NBUF, _CHUNK, N), jnp.float32),
            pltpu.SemaphoreType.DMA((_NBUF,)),
            pltpu.SemaphoreType.DMA((_NBUF,)),
            pltpu.VMEM((N, 1), jnp.float32),
            pltpu.VMEM((N, 1), jnp.float32),
        ],
    )(w, eye)


def _tail_body(w_ref, eye_ref, m_ref, s_ref, wt_main_ref, wt_ref, lse_ref):
    del wt_main_ref
    w = w_ref[...]
    wb = w.astype(jnp.bfloat16)
    wt_ref[...] = lax.dot_general(
        wb, eye_ref[...],
        dimension_numbers=(((0,), (0,)), ((), ())),
        preferred_element_type=jnp.float32,
    )
    col = _TAIL0 + lax.broadcasted_iota(jnp.int32, (N, _CHUNK), 1)
    x = jnp.where(col < M, w, _NEG)
    m_prev = m_ref[...]
    m_new = jnp.maximum(m_prev, jnp.max(x, axis=1, keepdims=True))
    s_new = (s_ref[...] * jnp.exp(m_prev - m_new)
             + jnp.sum(jnp.exp(x - m_new), axis=1, keepdims=True))
    lse_ref[...] = m_new + jnp.log(s_new)


def _tail_pass(w, eye, m1, s1, wt_main):
    return pl.pallas_call(
        _tail_body,
        grid=(1,),
        in_specs=[
            pl.BlockSpec((N, _CHUNK), lambda i: (0, _NCH)),
            pl.BlockSpec((N, N), lambda i: (0, 0)),
            pl.BlockSpec((N, 1), lambda i: (0, 0)),
            pl.BlockSpec((N, 1), lambda i: (0, 0)),
            pl.BlockSpec(memory_space=pl.ANY),
        ],
        input_output_aliases={4: 0},
        out_specs=[
            pl.BlockSpec((_CHUNK, N), lambda i: (_NCH, 0)),
            pl.BlockSpec((N, 1), lambda i: (0, 0)),
        ],
        out_shape=[
            jax.ShapeDtypeStruct((M, N), jnp.float32),
            jax.ShapeDtypeStruct((N, 1), jnp.float32),
        ],
    )(w, eye, m1, s1, wt_main)


_NC, _NS = 2, 16
_NW = _NC * _NS
_BPW = B // _NW
_GCHUNK = 128


def _gather_body(wt_hbm, idx_hbm, out_hbm, idx_v, rows_v, sem):
    wid = lax.axis_index("s") * _NC + lax.axis_index("c")
    base = wid * _BPW
    pltpu.sync_copy(idx_hbm.at[pl.ds(base, _BPW)], idx_v)
    for c in range(_BPW // _GCHUNK):
        pltpu.async_copy(
            wt_hbm.at[idx_v.at[pl.ds(c * _GCHUNK, _GCHUNK)]], rows_v, sem
        ).wait()
        pltpu.sync_copy(rows_v, out_hbm.at[pl.ds(base + c * _GCHUNK, _GCHUNK)])


def _sc_gather(wt, idx):
    mesh = plsc.VectorSubcoreMesh(core_axis_name="c", subcore_axis_name="s")
    k = functools.partial(
        pl.kernel,
        mesh=mesh,
        out_type=jax.ShapeDtypeStruct((B, N), jnp.float32),
        scratch_types=[
            pltpu.VMEM((_BPW,), jnp.int32),
            pltpu.VMEM((_GCHUNK, N), jnp.float32),
            pltpu.SemaphoreType.DMA,
        ],
    )(_gather_body)
    return k(wt, idx)


_CBLK = 2048


def _sub_body(g_ref, lse_ref, o_ref):
    o_ref[...] = g_ref[...] - lse_ref[...]


def _sub_lse(g, lse_row):
    return pl.pallas_call(
        _sub_body,
        grid=(B // _CBLK,),
        in_specs=[
            pl.BlockSpec((_CBLK, N), lambda i: (i, 0)),
            pl.BlockSpec((1, N), lambda i: (0, 0)),
        ],
        out_specs=pl.BlockSpec((_CBLK, N), lambda i: (i, 0)),
        out_shape=jax.ShapeDtypeStruct((B, N), jnp.float32),
    )(g, lse_row)


def kernel(x_t, W):
    idx = x_t.astype(jnp.int32)
    eye = jnp.eye(N, dtype=jnp.bfloat16)
    wt_main, m1, s1 = _main_pass(W, eye)
    wt, lse = _tail_pass(W, eye, m1, s1, wt_main)
    g = _sc_gather(wt, idx)
    lse_row = lse.reshape(1, N)
    return _sub_lse(g, lse_row)


# speedup vs baseline: 3.4355x; 2.8308x over previous
import functools

import jax
import jax.numpy as jnp
from jax import lax
from jax.experimental import pallas as pl
from jax.experimental.pallas import tpu as pltpu
from jax.experimental.pallas import tpu_sc as plsc

N = 256
M = 100000
B = 16384


_RCH = 2000
_NCH = M // _RCH
_NBUF = 8
_NEG = -1e30


def _in_copy(wt_hbm, inbuf, insem, g, slot):
    return pltpu.make_async_copy(
        wt_hbm.at[pl.ds(g * _RCH, _RCH)], inbuf.at[slot], insem.at[slot]
    )


def _lse_body(wt_hbm, lse_ref, inbuf, insem, m_ref, s_ref):
    m_ref[...] = jnp.full((1, N), _NEG, jnp.float32)
    s_ref[...] = jnp.zeros((1, N), jnp.float32)

    for g in range(_NBUF):
        _in_copy(wt_hbm, inbuf, insem, g, g).start()

    def body(g, carry):
        slot = lax.rem(g, _NBUF)
        _in_copy(wt_hbm, inbuf, insem, g, slot).wait()
        w = inbuf[slot]

        @pl.when(g + _NBUF < _NCH)
        def _refill():
            _in_copy(wt_hbm, inbuf, insem, g + _NBUF, slot).start()

        m_prev = m_ref[...]
        m_new = jnp.maximum(m_prev, jnp.max(w, axis=0, keepdims=True))
        s_ref[...] = (s_ref[...] * jnp.exp(m_prev - m_new)
                      + jnp.sum(jnp.exp(w - m_new), axis=0, keepdims=True))
        m_ref[...] = m_new
        return carry

    lax.fori_loop(0, _NCH, body, 0)
    lse_ref[...] = m_ref[...] + jnp.log(s_ref[...])


def _lse_pass(wt):
    return pl.pallas_call(
        _lse_body,
        in_specs=[pl.BlockSpec(memory_space=pl.ANY)],
        out_specs=pl.BlockSpec(memory_space=pltpu.VMEM),
        out_shape=jax.ShapeDtypeStruct((1, N), jnp.float32),
        scratch_shapes=[
            pltpu.VMEM((_NBUF, _RCH, N), jnp.float32),
            pltpu.SemaphoreType.DMA((_NBUF,)),
            pltpu.VMEM((1, N), jnp.float32),
            pltpu.VMEM((1, N), jnp.float32),
        ],
    )(wt)


_NC, _NS = 2, 16
_NW = _NC * _NS
_BPW = B // _NW
_GCHUNK = 128


def _gather_body(wt_hbm, idx_hbm, out_hbm, idx_v, rows_v, sem):
    wid = lax.axis_index("s") * _NC + lax.axis_index("c")
    base = wid * _BPW
    pltpu.sync_copy(idx_hbm.at[pl.ds(base, _BPW)], idx_v)
    for c in range(_BPW // _GCHUNK):
        pltpu.async_copy(
            wt_hbm.at[idx_v.at[pl.ds(c * _GCHUNK, _GCHUNK)]], rows_v, sem
        ).wait()
        pltpu.sync_copy(rows_v, out_hbm.at[pl.ds(base + c * _GCHUNK, _GCHUNK)])


def _sc_gather(wt, idx):
    mesh = plsc.VectorSubcoreMesh(core_axis_name="c", subcore_axis_name="s")
    k = functools.partial(
        pl.kernel,
        mesh=mesh,
        out_type=jax.ShapeDtypeStruct((B, N), jnp.float32),
        scratch_types=[
            pltpu.VMEM((_BPW,), jnp.int32),
            pltpu.VMEM((_GCHUNK, N), jnp.float32),
            pltpu.SemaphoreType.DMA,
        ],
    )(_gather_body)
    return k(wt, idx)


_CBLK = 2048


def _sub_body(g_ref, lse_ref, o_ref):
    o_ref[...] = g_ref[...] - lse_ref[...]


def _sub_lse(g, lse_row):
    return pl.pallas_call(
        _sub_body,
        grid=(B // _CBLK,),
        in_specs=[
            pl.BlockSpec((_CBLK, N), lambda i: (i, 0)),
            pl.BlockSpec((1, N), lambda i: (0, 0)),
        ],
        out_specs=pl.BlockSpec((_CBLK, N), lambda i: (i, 0)),
        out_shape=jax.ShapeDtypeStruct((B, N), jnp.float32),
    )(g, lse_row)


def kernel(x_t, W):
    idx = x_t.astype(jnp.int32)
    wt = W.T
    lse_row = _lse_pass(wt)
    g = _sc_gather(wt, idx)
    return _sub_lse(g, lse_row)
